# Initial kernel scaffold; baseline (speedup 1.0000x reference)
import jax
import jax.numpy as jnp
from jax.experimental import pallas as pl


def kernel(input_ids, word_table, pos_table, gamma, beta):
    raise NotImplementedError("write your pallas kernel here")


# speedup vs baseline: 1.8415x; 1.8415x over previous
import functools

import jax
import jax.numpy as jnp
from jax import lax
from jax.experimental import pallas as pl
from jax.experimental.pallas import tpu as pltpu
from jax.experimental.pallas import tpu_sc as plsc

DIM = 128
S = 200
EPS = 1e-12
L = 16
NV = DIM // L
NC = 2
NS = 16
NW = NC * NS
C = 128
G = 16
GPC = C // G


def _rsqrt(x):
    i = lax.bitcast_convert_type(x, jnp.int32)
    i = jnp.int32(0x5F3759DF) - lax.shift_right_logical(i, 1)
    y = lax.bitcast_convert_type(i, jnp.float32)
    for _ in range(3):
        y = y * (1.5 - 0.5 * x * y * y)
    return y


def _tree_add(xs):
    xs = list(xs)
    while len(xs) > 1:
        xs = [a + b for a, b in zip(xs[0::2], xs[1::2])]
    return xs[0]


def _ln_row(read_vec, write_vec, gvs, bvs):
    xs = [read_vec(j) for j in range(NV)]
    tot = _tree_add(xs)
    ssum = jnp.sum(tot)
    totsq = _tree_add([x * x for x in xs])
    ssq = jnp.sum(totsq)
    mean = ssum * (1.0 / DIM)
    var = ssq * (1.0 / DIM) - mean * mean
    sc = _rsqrt(var + EPS)
    for j in range(NV):
        write_vec(j, (xs[j] - mean) * sc * gvs[j] + bvs[j])


def _body(ids_hbm, table_hbm, pos_hbm, g_hbm, b_hbm, out_hbm,
          idx_v, pos_v, lnp_v, g_v, b_v, in_v, ou_v, gsem, ssem):
    n_rows = ids_hbm.shape[0]
    rw = n_rows // NW
    nchunk = rw // C
    wid = lax.axis_index("s") * NC + lax.axis_index("c")
    base = wid * rw

    pltpu.sync_copy(ids_hbm.at[pl.ds(base, rw)], idx_v)
    pltpu.sync_copy(pos_hbm.at[pl.ds(0, S)], pos_v)
    pltpu.sync_copy(g_hbm, g_v)
    pltpu.sync_copy(b_hbm, b_v)

    gvs = [g_v[pl.ds(j * L, L)] for j in range(NV)]
    bvs = [b_v[pl.ds(j * L, L)] for j in range(NV)]

    def lnp_body(i, _):
        _ln_row(lambda j: pos_v[i, pl.ds(j * L, L)],
                lambda j, v: lnp_v.__setitem__((i, pl.ds(j * L, L)), v),
                gvs, bvs)
        return 0
    lax.fori_loop(0, S, lnp_body, 0)

    lane = lax.iota(jnp.int32, L)

    def group_body(g, _):
        c = g // GPC
        r = g - c * GPC
        loc0 = c * C

        @pl.when(r == 0)
        def _gather():
            pltpu.async_copy(
                table_hbm.at[idx_v.at[pl.ds(loc0, C)]], in_v, gsem).wait()

        row0 = r * G
        for k in range(G):
            row = row0 + k
            loc = loc0 + row
            p = lax.rem(loc, S)
            _ln_row(lambda j: in_v[row, pl.ds(j * L, L)] + pos_v[p, pl.ds(j * L, L)],
                    lambda j, v: ou_v.__setitem__((row, pl.ds(j * L, L)), v),
                    gvs, bvs)

        idvec = idx_v[pl.ds(loc0 + row0, G)]
        nzero = jnp.sum(jnp.where(idvec == 0, 1, 0))

        @pl.when(nzero > 0)
        def _fix():
            def fix_body(k, _):
                row = row0 + k
                idk = jnp.sum(jnp.where(lane == k, idvec, 0))

                @pl.when(idk == 0)
                def _patch():
                    p = lax.rem(loc0 + row, S)
                    for j in range(NV):
                        ou_v[row, pl.ds(j * L, L)] = lnp_v[p, pl.ds(j * L, L)]
                return 0
            lax.fori_loop(0, G, fix_body, 0)

        @pl.when(r == GPC - 1)
        def _scatter():
            pltpu.sync_copy(ou_v, out_hbm.at[pl.ds(base + loc0, C)])
        return 0

    lax.fori_loop(0, nchunk * GPC, group_body, 0)


def kernel(input_ids, word_table, pos_table, gamma, beta):
    b, s = input_ids.shape
    n = b * s
    ids_flat = input_ids.reshape(n)
    mesh = plsc.VectorSubcoreMesh(core_axis_name="c", subcore_axis_name="s")
    f = functools.partial(
        pl.kernel,
        mesh=mesh,
        compiler_params=pltpu.CompilerParams(needs_layout_passes=False),
        out_type=jax.ShapeDtypeStruct((n, DIM), jnp.float32),
        scratch_types=[
            pltpu.VMEM((n // NW,), jnp.int32),
            pltpu.VMEM((S, DIM), jnp.float32),
            pltpu.VMEM((S, DIM), jnp.float32),
            pltpu.VMEM((DIM,), jnp.float32),
            pltpu.VMEM((DIM,), jnp.float32),
            pltpu.VMEM((C, DIM), jnp.float32),
            pltpu.VMEM((C, DIM), jnp.float32),
            pltpu.SemaphoreType.DMA,
            pltpu.SemaphoreType.DMA,
        ],
    )(_body)
    out = f(ids_flat, word_table, pos_table, gamma, beta)
    return out.reshape(b, s, DIM)


# speedup vs baseline: 2.2739x; 1.2349x over previous
import functools

import jax
import jax.numpy as jnp
from jax import lax
from jax.experimental import pallas as pl
from jax.experimental.pallas import tpu as pltpu
from jax.experimental.pallas import tpu_sc as plsc

DIM = 128
S = 200
EPS = 1e-12
L = 16
NV = DIM // L
NC = 2
NS = 16
NW = NC * NS
C = 128
G = 16
GPC = C // G


def _rsqrt(x):
    i = lax.bitcast_convert_type(x, jnp.int32)
    i = jnp.int32(0x5F3759DF) - lax.shift_right_logical(i, 1)
    y = lax.bitcast_convert_type(i, jnp.float32)
    for _ in range(3):
        y = y * (1.5 - 0.5 * x * y * y)
    return y


def _tree_add(xs):
    xs = list(xs)
    while len(xs) > 1:
        xs = [a + b for a, b in zip(xs[0::2], xs[1::2])]
    return xs[0]


def _ln_row(read_vec, write_vec, gvs, bvs):
    xs = [read_vec(j) for j in range(NV)]
    tot = _tree_add(xs)
    ssum = jnp.sum(tot)
    totsq = _tree_add([x * x for x in xs])
    ssq = jnp.sum(totsq)
    mean = ssum * (1.0 / DIM)
    var = ssq * (1.0 / DIM) - mean * mean
    sc = _rsqrt(var + EPS)
    for j in range(NV):
        write_vec(j, (xs[j] - mean) * sc * gvs[j] + bvs[j])


def _body(ids_hbm, table_hbm, pos_hbm, g_hbm, b_hbm, out_hbm,
          idx_v, pos_v, lnp_v, g_v, b_v, in_v, ou_v, gsem, ssem):
    n_rows = ids_hbm.shape[0]
    rw = n_rows // NW
    nchunk = rw // C
    wid = lax.axis_index("s") * NC + lax.axis_index("c")
    base = wid * rw

    pltpu.sync_copy(ids_hbm.at[pl.ds(base, rw)], idx_v)
    pltpu.sync_copy(pos_hbm.at[pl.ds(0, S)], pos_v)
    pltpu.sync_copy(g_hbm, g_v)
    pltpu.sync_copy(b_hbm, b_v)

    gvs = [g_v[pl.ds(j * L, L)] for j in range(NV)]
    bvs = [b_v[pl.ds(j * L, L)] for j in range(NV)]

    def lnp_body(i, _):
        _ln_row(lambda j: pos_v[i, pl.ds(j * L, L)],
                lambda j, v: lnp_v.__setitem__((i, pl.ds(j * L, L)), v),
                gvs, bvs)
        return 0
    lax.fori_loop(0, S, lnp_body, 0)

    lane = lax.iota(jnp.int32, L)

    def start_gather(c, b):
        pltpu.async_copy(
            table_hbm.at[idx_v.at[pl.ds(c * C, C)]], in_v.at[b], gsem.at[b])

    def start_scatter(c, b):
        pltpu.async_copy(
            ou_v.at[b], out_hbm.at[pl.ds(base + c * C, C)], ssem.at[b])

    def wait_gather(c, b):
        pltpu.make_async_copy(
            table_hbm.at[idx_v.at[pl.ds(c * C, C)]], in_v.at[b],
            gsem.at[b]).wait()

    def wait_scatter(c, b):
        pltpu.make_async_copy(
            ou_v.at[b], out_hbm.at[pl.ds(base + c * C, C)], ssem.at[b]).wait()

    start_gather(0, 0)
    start_gather(1, 1)

    def group_body(g, _):
        c = g // GPC
        r = g - c * GPC
        b = lax.rem(c, 2)
        loc0 = c * C

        @pl.when(r == 0)
        def _await_in():
            wait_gather(c, b)

        @pl.when((r == 0) & (c >= 2))
        def _await_out():
            wait_scatter(c - 2, b)

        row0 = r * G
        for k in range(G):
            row = row0 + k
            loc = loc0 + row
            p = lax.rem(loc, S)
            _ln_row(lambda j: in_v[b, row, pl.ds(j * L, L)] + pos_v[p, pl.ds(j * L, L)],
                    lambda j, v: ou_v.__setitem__((b, row, pl.ds(j * L, L)), v),
                    gvs, bvs)

        idvec = idx_v[pl.ds(loc0 + row0, G)]
        nzero = jnp.sum(jnp.where(idvec == 0, 1, 0))

        @pl.when(nzero > 0)
        def _fix():
            def fix_body(k, _):
                row = row0 + k
                idk = jnp.sum(jnp.where(lane == k, idvec, 0))

                @pl.when(idk == 0)
                def _patch():
                    p = lax.rem(loc0 + row, S)
                    for j in range(NV):
                        ou_v[b, row, pl.ds(j * L, L)] = lnp_v[p, pl.ds(j * L, L)]
                return 0
            lax.fori_loop(0, G, fix_body, 0)

        @pl.when(r == GPC - 1)
        def _flush():
            start_scatter(c, b)

            @pl.when(c + 2 < nchunk)
            def _next_gather():
                start_gather(c + 2, b)
        return 0

    lax.fori_loop(0, nchunk * GPC, group_body, 0)
    wait_scatter(nchunk - 2, 0)
    wait_scatter(nchunk - 1, 1)


def kernel(input_ids, word_table, pos_table, gamma, beta):
    b, s = input_ids.shape
    n = b * s
    ids_flat = input_ids.reshape(n)
    mesh = plsc.VectorSubcoreMesh(core_axis_name="c", subcore_axis_name="s")
    f = functools.partial(
        pl.kernel,
        mesh=mesh,
        compiler_params=pltpu.CompilerParams(needs_layout_passes=False),
        out_type=jax.ShapeDtypeStruct((n, DIM), jnp.float32),
        scratch_types=[
            pltpu.VMEM((n // NW,), jnp.int32),
            pltpu.VMEM((S, DIM), jnp.float32),
            pltpu.VMEM((S, DIM), jnp.float32),
            pltpu.VMEM((DIM,), jnp.float32),
            pltpu.VMEM((DIM,), jnp.float32),
            pltpu.VMEM((2, C, DIM), jnp.float32),
            pltpu.VMEM((2, C, DIM), jnp.float32),
            pltpu.SemaphoreType.DMA((2,)),
            pltpu.SemaphoreType.DMA((2,)),
        ],
    )(_body)
    out = f(ids_flat, word_table, pos_table, gamma, beta)
    return out.reshape(b, s, DIM)
